# baseline (speedup 1.0000x reference)
import jax
import jax.numpy as jnp
from jax import lax
from jax.experimental import pallas as pl
from jax.experimental.pallas import tpu as pltpu
from jax.experimental.pallas import tpu_sc as plsc

_B = 2097152
_NC = 2
_NS = 16
_NW = _NC * _NS
_PER_W = _B // _NW
_C = 4096
_H = _C // 2
_NCHUNK = _PER_W // _C
_GH = _H // 16
_TAB = 1024 * 1024
_TAB_SLICE = _TAB // _NS


def _sc_body(pv_hbm, table_hbm, bm_hbm, out_hbm,
             pos_v0, pos_v1, pos_v2, vel_v0, vel_v1, vel_v2,
             ida_v0, ida_v1, ida_v2, idb_v0, idb_v1, idb_v2,
             gta_v0, gta_v1, gta_v2, gtb_v0, gtb_v1, gtb_v2,
             bm_v, tab_sh, sem_p, sem_v, sem_ga, sem_gb, sem_oa, sem_ob):
    wid = lax.axis_index("s") * _NC + lax.axis_index("c")
    pltpu.sync_copy(bm_hbm, bm_v)
    b0 = bm_v[pl.ds(0, 16)]
    b1 = bm_v[pl.ds(16, 16)]
    m0 = bm_v[pl.ds(32, 16)]
    m1 = bm_v[pl.ds(48, 16)]
    w0 = wid * _PER_W
    pos_b = (pos_v0, pos_v1, pos_v2)
    vel_b = (vel_v0, vel_v1, vel_v2)
    ida_b = (ida_v0, ida_v1, ida_v2)
    idb_b = (idb_v0, idb_v1, idb_v2)
    gta_b = (gta_v0, gta_v1, gta_v2)
    gtb_b = (gtb_v0, gtb_v1, gtb_v2)

    def in_copies(ci):
        p = ci % 3
        s = pl.ds(w0 + ci * _C, _C)
        cp_p = pltpu.make_async_copy(pv_hbm.at[0, s], pos_b[p], sem_p.at[p])
        cp_v = pltpu.make_async_copy(pv_hbm.at[1, s], vel_b[p], sem_v.at[p])
        return cp_p, cp_v

    def gather_copies(ci):
        p = ci % 3
        ga = pltpu.make_async_copy(tab_sh.at[ida_b[p]], gta_b[p],
                                   sem_ga.at[p])
        gb = pltpu.make_async_copy(table_hbm.at[idb_b[p]], gtb_b[p],
                                   sem_gb.at[p])
        return ga, gb

    def out_copies(ci):
        p = ci % 3
        oa = pltpu.make_async_copy(gta_b[p],
                                   out_hbm.at[pl.ds(w0 + ci * _C, _H)],
                                   sem_oa.at[p])
        ob = pltpu.make_async_copy(gtb_b[p],
                                   out_hbm.at[pl.ds(w0 + ci * _C + _H, _H)],
                                   sem_ob.at[p])
        return oa, ob

    def compute(ci):
        p = ci % 3
        pos_r, vel_r = pos_b[p], vel_b[p]

        def quantize(s):
            pos = pos_r[s]
            vel = vel_r[s]
            row = ((pos + b0) * m0).astype(jnp.int32)
            col = ((vel + b1) * m1).astype(jnp.int32)
            return row * 1024 + col

        ida_r, idb_r = ida_b[p], idb_b[p]

        def body_a(g, c2):
            ida_r[pl.ds(g * 16, 16)] = quantize(pl.ds(g * 16, 16))
            return c2

        def body_b(g, c2):
            idb_r[pl.ds(g * 16, 16)] = quantize(pl.ds(_H + g * 16, 16))
            return c2

        lax.fori_loop(0, _GH, body_a, 0, unroll=8)
        lax.fori_loop(0, _GH, body_b, 0, unroll=8)

    for cj in range(3):
        cp_p, cp_v = in_copies(cj)
        cp_p.start()
        cp_v.start()
    sid = lax.axis_index("s")
    for t in range(_NS):
        @pl.when(sid == t)
        def _stage(t=t):
            ts = pl.ds(t * _TAB_SLICE, _TAB_SLICE)
            pltpu.sync_copy(table_hbm.at[ts], tab_sh.at[ts])
    plsc.subcore_barrier()
    for ci in range(_NCHUNK):
        cp_p, cp_v = in_copies(ci)
        cp_p.wait()
        cp_v.wait()
        compute(ci)
        if ci + 3 < _NCHUNK:
            cp_p, cp_v = in_copies(ci + 3)
            cp_p.start()
            cp_v.start()
        if ci >= 1:
            ga, gb = gather_copies(ci - 1)
            ga.wait()
            gb.wait()
            oa, ob = out_copies(ci - 1)
            oa.start()
            ob.start()
        if ci >= 2:
            oa, ob = out_copies(ci - 2)
            oa.wait()
            ob.wait()
        ga, gb = gather_copies(ci)
        ga.start()
        gb.start()
    ga, gb = gather_copies(_NCHUNK - 1)
    ga.wait()
    gb.wait()
    oa, ob = out_copies(_NCHUNK - 1)
    oa.start()
    ob.start()
    oa, ob = out_copies(_NCHUNK - 2)
    oa.wait()
    ob.wait()
    oa, ob = out_copies(_NCHUNK - 1)
    oa.wait()
    ob.wait()


def kernel(inp, data, b, m):
    pv = inp.T
    table = data.reshape(-1)
    bm = jnp.concatenate([
        jnp.broadcast_to(b[0], (16,)),
        jnp.broadcast_to(b[1], (16,)),
        jnp.broadcast_to(m[0], (16,)),
        jnp.broadcast_to(m[1], (16,)),
    ]).astype(jnp.float32)
    mesh = plsc.VectorSubcoreMesh(core_axis_name="c", subcore_axis_name="s",
                                  num_cores=_NC)
    return pl.kernel(
        _sc_body,
        out_type=jax.ShapeDtypeStruct((_B,), jnp.float32),
        mesh=mesh,
        scratch_types=[
            pltpu.VMEM((_C,), jnp.float32),
            pltpu.VMEM((_C,), jnp.float32),
            pltpu.VMEM((_C,), jnp.float32),
            pltpu.VMEM((_C,), jnp.float32),
            pltpu.VMEM((_C,), jnp.float32),
            pltpu.VMEM((_C,), jnp.float32),
            pltpu.VMEM((_H,), jnp.int32),
            pltpu.VMEM((_H,), jnp.int32),
            pltpu.VMEM((_H,), jnp.int32),
            pltpu.VMEM((_H,), jnp.int32),
            pltpu.VMEM((_H,), jnp.int32),
            pltpu.VMEM((_H,), jnp.int32),
            pltpu.VMEM((_H,), jnp.float32),
            pltpu.VMEM((_H,), jnp.float32),
            pltpu.VMEM((_H,), jnp.float32),
            pltpu.VMEM((_H,), jnp.float32),
            pltpu.VMEM((_H,), jnp.float32),
            pltpu.VMEM((_H,), jnp.float32),
            pltpu.VMEM((64,), jnp.float32),
            pltpu.VMEM_SHARED((_TAB,), jnp.float32),
            pltpu.SemaphoreType.DMA((3,)),
            pltpu.SemaphoreType.DMA((3,)),
            pltpu.SemaphoreType.DMA((3,)),
            pltpu.SemaphoreType.DMA((3,)),
            pltpu.SemaphoreType.DMA((3,)),
            pltpu.SemaphoreType.DMA((3,)),
        ],
    )(pv, table, bm)


# speedup vs baseline: 1.1413x; 1.1413x over previous
import jax
import jax.numpy as jnp
from jax import lax
from jax.experimental import pallas as pl
from jax.experimental.pallas import tpu as pltpu
from jax.experimental.pallas import tpu_sc as plsc

_B = 2097152
_NC = 2
_NS = 16
_NW = _NC * _NS
_PER_W = _B // _NW
_C = 4096
_HA = 3072
_HB = 1024
_NCHUNK = _PER_W // _C
_GA = _HA // 16
_GB = _HB // 16
_TAB = 1024 * 1024
_TAB_SLICE = _TAB // _NS


def _sc_body(pv_hbm, table_hbm, bm_hbm, out_hbm,
             pos_v0, pos_v1, pos_v2, vel_v0, vel_v1, vel_v2,
             ida_v0, ida_v1, ida_v2, idb_v0, idb_v1, idb_v2,
             gta_v0, gta_v1, gta_v2, gtb_v0, gtb_v1, gtb_v2,
             bm_v, tab_sh, sem_p, sem_v, sem_ga, sem_gb, sem_oa, sem_ob):
    wid = lax.axis_index("s") * _NC + lax.axis_index("c")
    pltpu.sync_copy(bm_hbm, bm_v)
    b0 = bm_v[pl.ds(0, 16)]
    b1 = bm_v[pl.ds(16, 16)]
    m0 = bm_v[pl.ds(32, 16)]
    m1 = bm_v[pl.ds(48, 16)]
    w0 = wid * _PER_W
    pos_b = (pos_v0, pos_v1, pos_v2)
    vel_b = (vel_v0, vel_v1, vel_v2)
    ida_b = (ida_v0, ida_v1, ida_v2)
    idb_b = (idb_v0, idb_v1, idb_v2)
    gta_b = (gta_v0, gta_v1, gta_v2)
    gtb_b = (gtb_v0, gtb_v1, gtb_v2)

    def in_copies(ci):
        p = ci % 3
        s = pl.ds(w0 + ci * _C, _C)
        cp_p = pltpu.make_async_copy(pv_hbm.at[0, s], pos_b[p], sem_p.at[p])
        cp_v = pltpu.make_async_copy(pv_hbm.at[1, s], vel_b[p], sem_v.at[p])
        return cp_p, cp_v

    def gather_copies(ci):
        p = ci % 3
        ga = pltpu.make_async_copy(tab_sh.at[ida_b[p]], gta_b[p],
                                   sem_ga.at[p])
        gb = pltpu.make_async_copy(table_hbm.at[idb_b[p]], gtb_b[p],
                                   sem_gb.at[p])
        return ga, gb

    def out_copies(ci):
        p = ci % 3
        oa = pltpu.make_async_copy(gta_b[p],
                                   out_hbm.at[pl.ds(w0 + ci * _C, _HA)],
                                   sem_oa.at[p])
        ob = pltpu.make_async_copy(gtb_b[p],
                                   out_hbm.at[pl.ds(w0 + ci * _C + _HA, _HB)],
                                   sem_ob.at[p])
        return oa, ob

    def compute(ci):
        p = ci % 3
        pos_r, vel_r = pos_b[p], vel_b[p]

        def quantize(s):
            pos = pos_r[s]
            vel = vel_r[s]
            row = ((pos + b0) * m0).astype(jnp.int32)
            col = ((vel + b1) * m1).astype(jnp.int32)
            return row * 1024 + col

        ida_r, idb_r = ida_b[p], idb_b[p]

        def body_a(g, c2):
            ida_r[pl.ds(g * 16, 16)] = quantize(pl.ds(g * 16, 16))
            return c2

        def body_b(g, c2):
            idb_r[pl.ds(g * 16, 16)] = quantize(pl.ds(_HA + g * 16, 16))
            return c2

        lax.fori_loop(0, _GA, body_a, 0, unroll=8)
        lax.fori_loop(0, _GB, body_b, 0, unroll=8)

    for cj in range(3):
        cp_p, cp_v = in_copies(cj)
        cp_p.start()
        cp_v.start()
    sid = lax.axis_index("s")
    for t in range(_NS):
        @pl.when(sid == t)
        def _stage(t=t):
            ts = pl.ds(t * _TAB_SLICE, _TAB_SLICE)
            pltpu.sync_copy(table_hbm.at[ts], tab_sh.at[ts])
    plsc.subcore_barrier()
    for ci in range(_NCHUNK):
        cp_p, cp_v = in_copies(ci)
        cp_p.wait()
        cp_v.wait()
        compute(ci)
        if ci + 3 < _NCHUNK:
            cp_p, cp_v = in_copies(ci + 3)
            cp_p.start()
            cp_v.start()
        if ci >= 1:
            ga, gb = gather_copies(ci - 1)
            ga.wait()
            gb.wait()
            oa, ob = out_copies(ci - 1)
            oa.start()
            ob.start()
        if ci >= 2:
            oa, ob = out_copies(ci - 2)
            oa.wait()
            ob.wait()
        ga, gb = gather_copies(ci)
        ga.start()
        gb.start()
    ga, gb = gather_copies(_NCHUNK - 1)
    ga.wait()
    gb.wait()
    oa, ob = out_copies(_NCHUNK - 1)
    oa.start()
    ob.start()
    oa, ob = out_copies(_NCHUNK - 2)
    oa.wait()
    ob.wait()
    oa, ob = out_copies(_NCHUNK - 1)
    oa.wait()
    ob.wait()


def kernel(inp, data, b, m):
    pv = inp.T
    table = data.reshape(-1)
    bm = jnp.concatenate([
        jnp.broadcast_to(b[0], (16,)),
        jnp.broadcast_to(b[1], (16,)),
        jnp.broadcast_to(m[0], (16,)),
        jnp.broadcast_to(m[1], (16,)),
    ]).astype(jnp.float32)
    mesh = plsc.VectorSubcoreMesh(core_axis_name="c", subcore_axis_name="s",
                                  num_cores=_NC)
    return pl.kernel(
        _sc_body,
        out_type=jax.ShapeDtypeStruct((_B,), jnp.float32),
        mesh=mesh,
        scratch_types=[
            pltpu.VMEM((_C,), jnp.float32),
            pltpu.VMEM((_C,), jnp.float32),
            pltpu.VMEM((_C,), jnp.float32),
            pltpu.VMEM((_C,), jnp.float32),
            pltpu.VMEM((_C,), jnp.float32),
            pltpu.VMEM((_C,), jnp.float32),
            pltpu.VMEM((_HA,), jnp.int32),
            pltpu.VMEM((_HA,), jnp.int32),
            pltpu.VMEM((_HA,), jnp.int32),
            pltpu.VMEM((_HB,), jnp.int32),
            pltpu.VMEM((_HB,), jnp.int32),
            pltpu.VMEM((_HB,), jnp.int32),
            pltpu.VMEM((_HA,), jnp.float32),
            pltpu.VMEM((_HA,), jnp.float32),
            pltpu.VMEM((_HA,), jnp.float32),
            pltpu.VMEM((_HB,), jnp.float32),
            pltpu.VMEM((_HB,), jnp.float32),
            pltpu.VMEM((_HB,), jnp.float32),
            pltpu.VMEM((64,), jnp.float32),
            pltpu.VMEM_SHARED((_TAB,), jnp.float32),
            pltpu.SemaphoreType.DMA((3,)),
            pltpu.SemaphoreType.DMA((3,)),
            pltpu.SemaphoreType.DMA((3,)),
            pltpu.SemaphoreType.DMA((3,)),
            pltpu.SemaphoreType.DMA((3,)),
            pltpu.SemaphoreType.DMA((3,)),
        ],
    )(pv, table, bm)


# speedup vs baseline: 1.2104x; 1.0605x over previous
import jax
import jax.numpy as jnp
from jax import lax
from jax.experimental import pallas as pl
from jax.experimental.pallas import tpu as pltpu
from jax.experimental.pallas import tpu_sc as plsc

_B = 2097152
_NC = 2
_NS = 16
_NW = _NC * _NS
_PER_W = _B // _NW
_C = 4096
_NCHUNK = _PER_W // _C
_G = _C // 16
_TAB = 1024 * 1024
_TAB_SLICE = _TAB // _NS


def _sc_body(pv_hbm, table_hbm, bm_hbm, out_hbm,
             pos_v0, pos_v1, pos_v2, vel_v0, vel_v1, vel_v2,
             idx_v0, idx_v1, idx_v2, gat_v0, gat_v1, gat_v2,
             bm_v, tab_sh, sem_p, sem_v, sem_g, sem_o):
    wid = lax.axis_index("s") * _NC + lax.axis_index("c")
    pltpu.sync_copy(bm_hbm, bm_v)
    b0 = bm_v[pl.ds(0, 16)]
    b1 = bm_v[pl.ds(16, 16)]
    m0 = bm_v[pl.ds(32, 16)]
    m1 = bm_v[pl.ds(48, 16)]
    w0 = wid * _PER_W
    pos_b = (pos_v0, pos_v1, pos_v2)
    vel_b = (vel_v0, vel_v1, vel_v2)
    idx_b = (idx_v0, idx_v1, idx_v2)
    gat_b = (gat_v0, gat_v1, gat_v2)

    def in_copies(ci):
        p = ci % 3
        s = pl.ds(w0 + ci * _C, _C)
        cp_p = pltpu.make_async_copy(pv_hbm.at[0, s], pos_b[p], sem_p.at[p])
        cp_v = pltpu.make_async_copy(pv_hbm.at[1, s], vel_b[p], sem_v.at[p])
        return cp_p, cp_v

    def gather_copy(ci):
        p = ci % 3
        return pltpu.make_async_copy(tab_sh.at[idx_b[p]], gat_b[p],
                                     sem_g.at[p])

    def out_copy(ci):
        p = ci % 3
        return pltpu.make_async_copy(gat_b[p],
                                     out_hbm.at[pl.ds(w0 + ci * _C, _C)],
                                     sem_o.at[p])

    def compute(ci):
        p = ci % 3
        pos_r, vel_r, idx_r = pos_b[p], vel_b[p], idx_b[p]

        def group_body(g, c2):
            s = pl.ds(g * 16, 16)
            pos = pos_r[s]
            vel = vel_r[s]
            row = ((pos + b0) * m0).astype(jnp.int32)
            col = ((vel + b1) * m1).astype(jnp.int32)
            idx_r[s] = row * 1024 + col
            return c2

        lax.fori_loop(0, _G, group_body, 0, unroll=8)

    for cj in range(3):
        cp_p, cp_v = in_copies(cj)
        cp_p.start()
        cp_v.start()
    sid = lax.axis_index("s")
    for t in range(_NS):
        @pl.when(sid == t)
        def _stage(t=t):
            ts = pl.ds(t * _TAB_SLICE, _TAB_SLICE)
            pltpu.sync_copy(table_hbm.at[ts], tab_sh.at[ts])
    plsc.subcore_barrier()
    for ci in range(_NCHUNK):
        cp_p, cp_v = in_copies(ci)
        cp_p.wait()
        cp_v.wait()
        compute(ci)
        if ci + 3 < _NCHUNK:
            cp_p, cp_v = in_copies(ci + 3)
            cp_p.start()
            cp_v.start()
        if ci >= 1:
            gather_copy(ci - 1).wait()
            out_copy(ci - 1).start()
        if ci >= 2:
            out_copy(ci - 2).wait()
        gather_copy(ci).start()
    gather_copy(_NCHUNK - 1).wait()
    out_copy(_NCHUNK - 1).start()
    out_copy(_NCHUNK - 2).wait()
    out_copy(_NCHUNK - 1).wait()


def kernel(inp, data, b, m):
    pv = inp.T
    table = data.reshape(-1)
    bm = jnp.concatenate([
        jnp.broadcast_to(b[0], (16,)),
        jnp.broadcast_to(b[1], (16,)),
        jnp.broadcast_to(m[0], (16,)),
        jnp.broadcast_to(m[1], (16,)),
    ]).astype(jnp.float32)
    mesh = plsc.VectorSubcoreMesh(core_axis_name="c", subcore_axis_name="s",
                                  num_cores=_NC)
    return pl.kernel(
        _sc_body,
        out_type=jax.ShapeDtypeStruct((_B,), jnp.float32),
        mesh=mesh,
        scratch_types=[
            pltpu.VMEM((_C,), jnp.float32),
            pltpu.VMEM((_C,), jnp.float32),
            pltpu.VMEM((_C,), jnp.float32),
            pltpu.VMEM((_C,), jnp.float32),
            pltpu.VMEM((_C,), jnp.float32),
            pltpu.VMEM((_C,), jnp.float32),
            pltpu.VMEM((_C,), jnp.int32),
            pltpu.VMEM((_C,), jnp.int32),
            pltpu.VMEM((_C,), jnp.int32),
            pltpu.VMEM((_C,), jnp.float32),
            pltpu.VMEM((_C,), jnp.float32),
            pltpu.VMEM((_C,), jnp.float32),
            pltpu.VMEM((64,), jnp.float32),
            pltpu.VMEM_SHARED((_TAB,), jnp.float32),
            pltpu.SemaphoreType.DMA((3,)),
            pltpu.SemaphoreType.DMA((3,)),
            pltpu.SemaphoreType.DMA((3,)),
            pltpu.SemaphoreType.DMA((3,)),
        ],
    )(pv, table, bm)
